# Initial kernel scaffold; baseline (speedup 1.0000x reference)
import jax
import jax.numpy as jnp
from jax.experimental import pallas as pl


def kernel(x, edge_index, W_neigh, W_root, b_sage, gamma, beta, Wg1, bg1, Wg2, bg2, Wp, bp):
    raise NotImplementedError("write your pallas kernel here")


# speedup vs baseline: 3.8710x; 3.8710x over previous
import functools

import jax
import jax.numpy as jnp
from jax import lax
from jax.experimental import pallas as pl
from jax.experimental.pallas import tpu as pltpu
from jax.experimental.pallas import tpu_sc as plsc

N = 10000
E = 160000
D_IN = 256
D_OUT = 256
D_H = 128
D_P = 64

NC = 2
NS = 16
DH = D_IN // NC
EPT = E // NS
CHUNK = 80
NCHUNK = EPT // CHUNK
ROWS_T = 10240 // NS
NPAD = 10240
CPT = NPAD // NS


def _sc_body(x2, src, dst, z2, z1, agg_out, cnt_out,
             idx_v, dst_v, rows_v, ones_v, acc_sh, cnt_sh, sem):
    cid = lax.axis_index("c")
    sid = lax.axis_index("s")

    pltpu.sync_copy(z2.at[pl.ds(sid * ROWS_T, ROWS_T)],
                    acc_sh.at[pl.ds(sid * ROWS_T, ROWS_T)])

    @pl.when(cid == 0)
    def _():
        pltpu.sync_copy(z1.at[pl.ds(sid * CPT, CPT)],
                        cnt_sh.at[pl.ds(sid * CPT, CPT)])

    for j in range(CHUNK // 16):
        ones_v[pl.ds(j * 16, 16)] = jnp.ones((16,), jnp.float32)

    plsc.subcore_barrier()

    def step(k, _):
        base = sid * EPT + k * CHUNK
        pltpu.sync_copy(src.at[pl.ds(base, CHUNK)], idx_v)
        pltpu.sync_copy(dst.at[pl.ds(base, CHUNK)], dst_v)
        for j in range(CHUNK // 16):
            v = idx_v[pl.ds(j * 16, 16)]
            idx_v[pl.ds(j * 16, 16)] = v * 2 + cid
        pltpu.async_copy(x2.at[idx_v], rows_v, sem).wait()
        pltpu.sync_copy(rows_v, acc_sh.at[dst_v], add=True)

        @pl.when(cid == 0)
        def _():
            pltpu.sync_copy(ones_v, cnt_sh.at[dst_v], add=True)

        return _

    lax.fori_loop(0, NCHUNK, step, None)

    plsc.subcore_barrier()

    pltpu.sync_copy(acc_sh.at[pl.ds(sid * ROWS_T, ROWS_T)],
                    agg_out.at[cid, pl.ds(sid * ROWS_T, ROWS_T)])

    @pl.when(cid == 0)
    def _():
        pltpu.sync_copy(cnt_sh.at[pl.ds(sid * CPT, CPT)],
                        cnt_out.at[pl.ds(sid * CPT, CPT)])


def _sc_segment_sum(x2, src, dst):
    z2 = jnp.zeros((NPAD, DH), jnp.float32)
    z1 = jnp.zeros((NPAD,), jnp.float32)
    mesh = plsc.VectorSubcoreMesh(core_axis_name="c", subcore_axis_name="s")
    k = pl.kernel(
        _sc_body,
        out_type=(jax.ShapeDtypeStruct((NC, NPAD, DH), jnp.float32),
                  jax.ShapeDtypeStruct((NPAD,), jnp.float32)),
        mesh=mesh,
        scratch_types=[
            pltpu.VMEM((CHUNK,), jnp.int32),
            pltpu.VMEM((CHUNK,), jnp.int32),
            pltpu.VMEM((CHUNK, DH), jnp.float32),
            pltpu.VMEM((CHUNK,), jnp.float32),
            pltpu.VMEM_SHARED((NPAD, DH), jnp.float32),
            pltpu.VMEM_SHARED((NPAD,), jnp.float32),
            pltpu.SemaphoreType.DMA,
        ],
    )
    return k(x2, src, dst, z2, z1)


ROW_BLK = 1000


def _dense_body(x_ref, a0_ref, a1_ref, cnt_ref, wn0_ref, wn1_ref, wr_ref,
                b_ref, gam_ref, bet_ref, wg1_ref, bg1_ref, wg2_ref, bg2_ref,
                wp_ref, bp_ref, g_ref, xp_ref):
    inv = 1.0 / jnp.maximum(cnt_ref[...], 1.0)
    m0 = a0_ref[...] * inv
    m1 = a1_ref[...] * inv
    dims = (((1,), (1,)), ((), ()))
    h = (lax.dot_general(m0, wn0_ref[...], dims, preferred_element_type=jnp.float32)
         + lax.dot_general(m1, wn1_ref[...], dims, preferred_element_type=jnp.float32)
         + lax.dot_general(x_ref[...], wr_ref[...], dims, preferred_element_type=jnp.float32)
         + b_ref[...])
    h = jnp.maximum(h, 0.0)
    mu = jnp.mean(h, axis=-1, keepdims=True)
    hc = h - mu
    var = jnp.mean(hc * hc, axis=-1, keepdims=True)
    h = hc * lax.rsqrt(var + 1e-5) * gam_ref[...] + bet_ref[...]
    hg = jnp.maximum(
        lax.dot_general(h, wg1_ref[...], dims, preferred_element_type=jnp.float32)
        + bg1_ref[...], 0.0)
    g_ref[...] = (lax.dot_general(hg, wg2_ref[...], dims,
                                  preferred_element_type=jnp.float32)
                  + bg2_ref[...])
    xp_ref[...] = jnp.maximum(
        lax.dot_general(h, wp_ref[...], dims, preferred_element_type=jnp.float32)
        + bp_ref[...], 0.0)


def _finish_body(g_ref, xp_ref, out_ref):
    g = g_ref[...]
    m = jnp.max(g, axis=0, keepdims=True)
    e = jnp.exp(g - m)
    s = jnp.sum(e, axis=0, keepdims=True)
    o = (e / s) * xp_ref[...]
    nrm = jnp.sqrt(jnp.sum(o * o, axis=1, keepdims=True))
    out_ref[...] = o / jnp.maximum(nrm, 1e-12)


def kernel(x, edge_index, W_neigh, W_root, b_sage, gamma, beta,
           Wg1, bg1, Wg2, bg2, Wp, bp):
    src = edge_index[0].astype(jnp.int32)
    dst = edge_index[1].astype(jnp.int32)
    x2 = x.reshape(2 * N, DH)

    agg, cnt = _sc_segment_sum(x2, src, dst)

    grid = N // ROW_BLK
    row = lambda i: (i, 0)
    full = lambda i: (0, 0)
    g, xp = pl.pallas_call(
        _dense_body,
        grid=(grid,),
        in_specs=[
            pl.BlockSpec((ROW_BLK, D_IN), row),
            pl.BlockSpec((ROW_BLK, DH), row),
            pl.BlockSpec((ROW_BLK, DH), row),
            pl.BlockSpec((ROW_BLK, 1), row),
            pl.BlockSpec((D_OUT, DH), full),
            pl.BlockSpec((D_OUT, DH), full),
            pl.BlockSpec((D_OUT, D_IN), full),
            pl.BlockSpec((1, D_OUT), full),
            pl.BlockSpec((1, D_OUT), full),
            pl.BlockSpec((1, D_OUT), full),
            pl.BlockSpec((D_H, D_OUT), full),
            pl.BlockSpec((1, D_H), full),
            pl.BlockSpec((D_P, D_H), full),
            pl.BlockSpec((1, D_P), full),
            pl.BlockSpec((D_P, D_OUT), full),
            pl.BlockSpec((1, D_P), full),
        ],
        out_specs=[
            pl.BlockSpec((ROW_BLK, D_P), row),
            pl.BlockSpec((ROW_BLK, D_P), row),
        ],
        out_shape=[
            jax.ShapeDtypeStruct((N, D_P), jnp.float32),
            jax.ShapeDtypeStruct((N, D_P), jnp.float32),
        ],
    )(x, agg[0, :N], agg[1, :N], cnt[:N].reshape(N, 1),
      W_neigh[:, :DH], W_neigh[:, DH:], W_root,
      b_sage.reshape(1, -1), gamma.reshape(1, -1), beta.reshape(1, -1),
      Wg1, bg1.reshape(1, -1), Wg2, bg2.reshape(1, -1),
      Wp, bp.reshape(1, -1))

    out = pl.pallas_call(
        _finish_body,
        out_shape=jax.ShapeDtypeStruct((N, D_P), jnp.float32),
    )(g, xp)
    return out


# speedup vs baseline: 4.4020x; 1.1372x over previous
import functools

import jax
import jax.numpy as jnp
from jax import lax
from jax.experimental import pallas as pl
from jax.experimental.pallas import tpu as pltpu
from jax.experimental.pallas import tpu_sc as plsc

N = 10000
E = 160000
D_IN = 256
D_OUT = 256
D_H = 128
D_P = 64

NC = 2
NS = 16
DH = D_IN // NC
EPT = E // NS
CHUNK = 80
NCHUNK = EPT // CHUNK
ROWS_T = 10240 // NS
NPAD = 10240
CPT = NPAD // NS


U = 1
NITER = NCHUNK // U


def _sc_body(x2, src, dst, z2, z1, agg_out, cnt_out,
             s_v, d_v, rows_v, ones_v, acc_sh, cnt_sh, sem_i, sem_g, sem_s):
    cid = lax.axis_index("c")
    sid = lax.axis_index("s")

    def idx_copies(t, u):
        base = sid * EPT + (t * U + u) * CHUNK
        return (pltpu.make_async_copy(src.at[pl.ds(base, CHUNK)], s_v.at[u], sem_i.at[u]),
                pltpu.make_async_copy(dst.at[pl.ds(base, CHUNK)], d_v.at[u], sem_i.at[u]))

    for u in range(U):
        a, b = idx_copies(0, u)
        a.start()
        b.start()

    pltpu.sync_copy(z2.at[pl.ds(sid * ROWS_T, ROWS_T)],
                    acc_sh.at[pl.ds(sid * ROWS_T, ROWS_T)])

    @pl.when(cid == 0)
    def _():
        pltpu.sync_copy(z1.at[pl.ds(sid * CPT, CPT)],
                        cnt_sh.at[pl.ds(sid * CPT, CPT)])

    for j in range(CHUNK // 16):
        ones_v[pl.ds(j * 16, 16)] = jnp.ones((16,), jnp.float32)

    plsc.subcore_barrier()

    def step(t, _):
        for u in range(U):
            a, b = idx_copies(t, u)
            a.wait()
            b.wait()
        for u in range(U):
            for j in range(CHUNK // 16):
                v = s_v[u, pl.ds(j * 16, 16)]
                s_v[u, pl.ds(j * 16, 16)] = v * 2 + cid
        for u in range(U):
            pltpu.async_copy(x2.at[s_v.at[u]], rows_v.at[u], sem_g.at[u])
        for u in range(U):
            pltpu.make_async_copy(x2.at[s_v.at[u]], rows_v.at[u], sem_g.at[u]).wait()
        for u in range(U):
            pltpu.async_copy(rows_v.at[u], acc_sh.at[d_v.at[u]], sem_s.at[u], add=True)
            pltpu.make_async_copy(rows_v.at[u], acc_sh.at[d_v.at[u]], sem_s.at[u]).wait()

            @pl.when(cid == 0)
            def _():
                pltpu.async_copy(ones_v, cnt_sh.at[d_v.at[u]], sem_s.at[u], add=True)
                pltpu.make_async_copy(ones_v, cnt_sh.at[d_v.at[u]], sem_s.at[u]).wait()

        @pl.when(t + 1 < NITER)
        def _():
            for u in range(U):
                a, b = idx_copies(t + 1, u)
                a.start()
                b.start()

        return _

    lax.fori_loop(0, NITER, step, None)

    plsc.subcore_barrier()

    pltpu.sync_copy(acc_sh.at[pl.ds(sid * ROWS_T, ROWS_T)],
                    agg_out.at[cid, pl.ds(sid * ROWS_T, ROWS_T)])

    @pl.when(cid == 0)
    def _():
        pltpu.sync_copy(cnt_sh.at[pl.ds(sid * CPT, CPT)],
                        cnt_out.at[pl.ds(sid * CPT, CPT)])


def _sc_segment_sum(x2, src, dst):
    z2 = jnp.zeros((NPAD, DH), jnp.float32)
    z1 = jnp.zeros((NPAD,), jnp.float32)
    mesh = plsc.VectorSubcoreMesh(core_axis_name="c", subcore_axis_name="s")
    k = pl.kernel(
        _sc_body,
        out_type=(jax.ShapeDtypeStruct((NC, NPAD, DH), jnp.float32),
                  jax.ShapeDtypeStruct((NPAD,), jnp.float32)),
        mesh=mesh,
        scratch_types=[
            pltpu.VMEM((U, CHUNK), jnp.int32),
            pltpu.VMEM((U, CHUNK), jnp.int32),
            pltpu.VMEM((U, CHUNK, DH), jnp.float32),
            pltpu.VMEM((CHUNK,), jnp.float32),
            pltpu.VMEM_SHARED((NPAD, DH), jnp.float32),
            pltpu.VMEM_SHARED((NPAD,), jnp.float32),
            pltpu.SemaphoreType.DMA((U,)),
            pltpu.SemaphoreType.DMA((U,)),
            pltpu.SemaphoreType.DMA((U,)),
        ],
    )
    return k(x2, src, dst, z2, z1)


ROW_BLK = 1000


def _dense_body(x_ref, a0_ref, a1_ref, cnt_ref, wn0_ref, wn1_ref, wr_ref,
                b_ref, gam_ref, bet_ref, wg1_ref, bg1_ref, wg2_ref, bg2_ref,
                wp_ref, bp_ref, g_ref, xp_ref):
    inv = 1.0 / jnp.maximum(cnt_ref[...], 1.0)
    m0 = a0_ref[...] * inv
    m1 = a1_ref[...] * inv
    dims = (((1,), (1,)), ((), ()))
    h = (lax.dot_general(m0, wn0_ref[...], dims, preferred_element_type=jnp.float32)
         + lax.dot_general(m1, wn1_ref[...], dims, preferred_element_type=jnp.float32)
         + lax.dot_general(x_ref[...], wr_ref[...], dims, preferred_element_type=jnp.float32)
         + b_ref[...])
    h = jnp.maximum(h, 0.0)
    mu = jnp.mean(h, axis=-1, keepdims=True)
    hc = h - mu
    var = jnp.mean(hc * hc, axis=-1, keepdims=True)
    h = hc * lax.rsqrt(var + 1e-5) * gam_ref[...] + bet_ref[...]
    hg = jnp.maximum(
        lax.dot_general(h, wg1_ref[...], dims, preferred_element_type=jnp.float32)
        + bg1_ref[...], 0.0)
    g_ref[...] = (lax.dot_general(hg, wg2_ref[...], dims,
                                  preferred_element_type=jnp.float32)
                  + bg2_ref[...])
    xp_ref[...] = jnp.maximum(
        lax.dot_general(h, wp_ref[...], dims, preferred_element_type=jnp.float32)
        + bp_ref[...], 0.0)


def _finish_body(g_ref, xp_ref, out_ref):
    g = g_ref[...]
    m = jnp.max(g, axis=0, keepdims=True)
    e = jnp.exp(g - m)
    s = jnp.sum(e, axis=0, keepdims=True)
    o = (e / s) * xp_ref[...]
    nrm = jnp.sqrt(jnp.sum(o * o, axis=1, keepdims=True))
    out_ref[...] = o / jnp.maximum(nrm, 1e-12)


def kernel(x, edge_index, W_neigh, W_root, b_sage, gamma, beta,
           Wg1, bg1, Wg2, bg2, Wp, bp):
    src = edge_index[0].astype(jnp.int32)
    dst = edge_index[1].astype(jnp.int32)
    x2 = x.reshape(2 * N, DH)

    agg, cnt = _sc_segment_sum(x2, src, dst)

    grid = N // ROW_BLK
    row = lambda i: (i, 0)
    full = lambda i: (0, 0)
    g, xp = pl.pallas_call(
        _dense_body,
        grid=(grid,),
        in_specs=[
            pl.BlockSpec((ROW_BLK, D_IN), row),
            pl.BlockSpec((ROW_BLK, DH), row),
            pl.BlockSpec((ROW_BLK, DH), row),
            pl.BlockSpec((ROW_BLK, 1), row),
            pl.BlockSpec((D_OUT, DH), full),
            pl.BlockSpec((D_OUT, DH), full),
            pl.BlockSpec((D_OUT, D_IN), full),
            pl.BlockSpec((1, D_OUT), full),
            pl.BlockSpec((1, D_OUT), full),
            pl.BlockSpec((1, D_OUT), full),
            pl.BlockSpec((D_H, D_OUT), full),
            pl.BlockSpec((1, D_H), full),
            pl.BlockSpec((D_P, D_H), full),
            pl.BlockSpec((1, D_P), full),
            pl.BlockSpec((D_P, D_OUT), full),
            pl.BlockSpec((1, D_P), full),
        ],
        out_specs=[
            pl.BlockSpec((ROW_BLK, D_P), row),
            pl.BlockSpec((ROW_BLK, D_P), row),
        ],
        out_shape=[
            jax.ShapeDtypeStruct((N, D_P), jnp.float32),
            jax.ShapeDtypeStruct((N, D_P), jnp.float32),
        ],
    )(x, agg[0, :N], agg[1, :N], cnt[:N].reshape(N, 1),
      W_neigh[:, :DH], W_neigh[:, DH:], W_root,
      b_sage.reshape(1, -1), gamma.reshape(1, -1), beta.reshape(1, -1),
      Wg1, bg1.reshape(1, -1), Wg2, bg2.reshape(1, -1),
      Wp, bp.reshape(1, -1))

    out = pl.pallas_call(
        _finish_body,
        out_shape=jax.ShapeDtypeStruct((N, D_P), jnp.float32),
    )(g, xp)
    return out


# speedup vs baseline: 4.7966x; 1.0896x over previous
import functools

import jax
import jax.numpy as jnp
from jax import lax
from jax.experimental import pallas as pl
from jax.experimental.pallas import tpu as pltpu
from jax.experimental.pallas import tpu_sc as plsc

N = 10000
E = 160000
D_IN = 256
D_OUT = 256
D_H = 128
D_P = 64

NC = 2
NS = 16
DH = D_IN // NC
EPT = 10240
CHUNK = 80
NCHUNK = EPT // CHUNK
ROWS_T = 10240 // NS
NPAD = 10240
CPT = NPAD // NS


U = 4
NITER = NCHUNK // U


def _sc_body(x2, src, dst, z2, z1, agg_out, cnt_out, *scr):
    sv = scr[0:U]
    dv = scr[U:2 * U]
    rv = scr[2 * U:3 * U]
    ones_v, acc_sh, cnt_sh, sem_i, sem_g, sem_s = scr[3 * U:]
    cid = lax.axis_index("c")
    sid = lax.axis_index("s")

    def idx_copies(t, u):
        base = sid * EPT + (t * U + u) * CHUNK
        return (pltpu.make_async_copy(src.at[pl.ds(base, CHUNK)], sv[u], sem_i.at[u]),
                pltpu.make_async_copy(dst.at[pl.ds(base, CHUNK)], dv[u], sem_i.at[u]))

    for u in range(U):
        a, b = idx_copies(0, u)
        a.start()
        b.start()

    pltpu.sync_copy(z2.at[pl.ds(sid * ROWS_T, ROWS_T)],
                    acc_sh.at[pl.ds(sid * ROWS_T, ROWS_T)])

    @pl.when(cid == 0)
    def _():
        pltpu.sync_copy(z1.at[pl.ds(sid * CPT, CPT)],
                        cnt_sh.at[pl.ds(sid * CPT, CPT)])

    for j in range(CHUNK // 16):
        ones_v[pl.ds(j * 16, 16)] = jnp.ones((16,), jnp.float32)

    plsc.subcore_barrier()

    def step(t, _):
        for u in range(U):
            a, b = idx_copies(t, u)
            a.wait()
            b.wait()
        for u in range(U):
            for j in range(CHUNK // 16):
                v = sv[u][pl.ds(j * 16, 16)]
                sv[u][pl.ds(j * 16, 16)] = v * 2 + cid
        for u in range(U):
            pltpu.async_copy(x2.at[sv[u]], rv[u], sem_g.at[u])
            pltpu.make_async_copy(x2.at[sv[u]], rv[u], sem_g.at[u]).wait()
        for u in range(U):
            pltpu.async_copy(rv[u], acc_sh.at[dv[u]], sem_s.at[u], add=True)
            pltpu.make_async_copy(rv[u], acc_sh.at[dv[u]], sem_s.at[u]).wait()

            @pl.when(cid == 0)
            def _():
                pltpu.async_copy(ones_v, cnt_sh.at[dv[u]], sem_s.at[u], add=True)
                pltpu.make_async_copy(ones_v, cnt_sh.at[dv[u]], sem_s.at[u]).wait()

        @pl.when(t + 1 < NITER)
        def _():
            for u in range(U):
                a, b = idx_copies(t + 1, u)
                a.start()
                b.start()

        return _

    lax.fori_loop(0, NITER, step, None)

    plsc.subcore_barrier()

    pltpu.sync_copy(acc_sh.at[pl.ds(sid * ROWS_T, ROWS_T)],
                    agg_out.at[cid, pl.ds(sid * ROWS_T, ROWS_T)])

    @pl.when(cid == 0)
    def _():
        pltpu.sync_copy(cnt_sh.at[pl.ds(sid * CPT, CPT)],
                        cnt_out.at[pl.ds(sid * CPT, CPT)])


def _sc_segment_sum(x2, src, dst):
    z2 = jnp.zeros((NPAD, DH), jnp.float32)
    z1 = jnp.zeros((NPAD,), jnp.float32)
    mesh = plsc.VectorSubcoreMesh(core_axis_name="c", subcore_axis_name="s")
    k = pl.kernel(
        _sc_body,
        out_type=(jax.ShapeDtypeStruct((NC, NPAD, DH), jnp.float32),
                  jax.ShapeDtypeStruct((NPAD,), jnp.float32)),
        mesh=mesh,
        scratch_types=(
            [pltpu.VMEM((CHUNK,), jnp.int32) for _ in range(U)]
            + [pltpu.VMEM((CHUNK,), jnp.int32) for _ in range(U)]
            + [pltpu.VMEM((CHUNK, DH), jnp.float32) for _ in range(U)]
            + [
            pltpu.VMEM((CHUNK,), jnp.float32),
            pltpu.VMEM_SHARED((NPAD, DH), jnp.float32),
            pltpu.VMEM_SHARED((NPAD,), jnp.float32),
            pltpu.SemaphoreType.DMA((U,)),
            pltpu.SemaphoreType.DMA((U,)),
            pltpu.SemaphoreType.DMA((U,)),
        ]),
    )
    return k(x2, src, dst, z2, z1)


ROW_BLK = 1000


def _dense_body(x_ref, a0_ref, a1_ref, cnt_ref, wn0_ref, wn1_ref, wr_ref,
                b_ref, gam_ref, bet_ref, wg1_ref, bg1_ref, wg2_ref, bg2_ref,
                wp_ref, bp_ref, g_ref, xp_ref):
    inv = 1.0 / jnp.maximum(cnt_ref[...], 1.0)
    m0 = a0_ref[...] * inv
    m1 = a1_ref[...] * inv
    dims = (((1,), (1,)), ((), ()))
    h = (lax.dot_general(m0, wn0_ref[...], dims, preferred_element_type=jnp.float32)
         + lax.dot_general(m1, wn1_ref[...], dims, preferred_element_type=jnp.float32)
         + lax.dot_general(x_ref[...], wr_ref[...], dims, preferred_element_type=jnp.float32)
         + b_ref[...])
    h = jnp.maximum(h, 0.0)
    mu = jnp.mean(h, axis=-1, keepdims=True)
    hc = h - mu
    var = jnp.mean(hc * hc, axis=-1, keepdims=True)
    h = hc * lax.rsqrt(var + 1e-5) * gam_ref[...] + bet_ref[...]
    hg = jnp.maximum(
        lax.dot_general(h, wg1_ref[...], dims, preferred_element_type=jnp.float32)
        + bg1_ref[...], 0.0)
    g_ref[...] = (lax.dot_general(hg, wg2_ref[...], dims,
                                  preferred_element_type=jnp.float32)
                  + bg2_ref[...])
    xp_ref[...] = jnp.maximum(
        lax.dot_general(h, wp_ref[...], dims, preferred_element_type=jnp.float32)
        + bp_ref[...], 0.0)


def _finish_body(g_ref, xp_ref, out_ref):
    g = g_ref[...]
    m = jnp.max(g, axis=0, keepdims=True)
    e = jnp.exp(g - m)
    s = jnp.sum(e, axis=0, keepdims=True)
    o = (e / s) * xp_ref[...]
    nrm = jnp.sqrt(jnp.sum(o * o, axis=1, keepdims=True))
    out_ref[...] = o / jnp.maximum(nrm, 1e-12)


def kernel(x, edge_index, W_neigh, W_root, b_sage, gamma, beta,
           Wg1, bg1, Wg2, bg2, Wp, bp):
    src = edge_index[0].astype(jnp.int32)
    dst = edge_index[1].astype(jnp.int32)
    x2 = x.reshape(2 * N, DH)

    npe = EPT - E // NS
    pad_s = (jnp.arange(npe, dtype=jnp.int32)[None, :] * 41
             + jnp.arange(NS, dtype=jnp.int32)[:, None] * 13) % N
    pad_d = (N + jnp.arange(npe, dtype=jnp.int32)[None, :]
             + jnp.zeros((NS, 1), jnp.int32))
    src_p = jnp.concatenate([src.reshape(NS, -1), pad_s], axis=1).reshape(-1)
    dst_p = jnp.concatenate([dst.reshape(NS, -1), pad_d], axis=1).reshape(-1)

    agg, cnt = _sc_segment_sum(x2, src_p, dst_p)

    grid = N // ROW_BLK
    row = lambda i: (i, 0)
    full = lambda i: (0, 0)
    g, xp = pl.pallas_call(
        _dense_body,
        grid=(grid,),
        in_specs=[
            pl.BlockSpec((ROW_BLK, D_IN), row),
            pl.BlockSpec((ROW_BLK, DH), row),
            pl.BlockSpec((ROW_BLK, DH), row),
            pl.BlockSpec((ROW_BLK, 1), row),
            pl.BlockSpec((D_OUT, DH), full),
            pl.BlockSpec((D_OUT, DH), full),
            pl.BlockSpec((D_OUT, D_IN), full),
            pl.BlockSpec((1, D_OUT), full),
            pl.BlockSpec((1, D_OUT), full),
            pl.BlockSpec((1, D_OUT), full),
            pl.BlockSpec((D_H, D_OUT), full),
            pl.BlockSpec((1, D_H), full),
            pl.BlockSpec((D_P, D_H), full),
            pl.BlockSpec((1, D_P), full),
            pl.BlockSpec((D_P, D_OUT), full),
            pl.BlockSpec((1, D_P), full),
        ],
        out_specs=[
            pl.BlockSpec((ROW_BLK, D_P), row),
            pl.BlockSpec((ROW_BLK, D_P), row),
        ],
        out_shape=[
            jax.ShapeDtypeStruct((N, D_P), jnp.float32),
            jax.ShapeDtypeStruct((N, D_P), jnp.float32),
        ],
    )(x, agg[0, :N], agg[1, :N], cnt[:N].reshape(N, 1),
      W_neigh[:, :DH], W_neigh[:, DH:], W_root,
      b_sage.reshape(1, -1), gamma.reshape(1, -1), beta.reshape(1, -1),
      Wg1, bg1.reshape(1, -1), Wg2, bg2.reshape(1, -1),
      Wp, bp.reshape(1, -1))

    out = pl.pallas_call(
        _finish_body,
        out_shape=jax.ShapeDtypeStruct((N, D_P), jnp.float32),
    )(g, xp)
    return out


# speedup vs baseline: 6.1999x; 1.2925x over previous
import functools

import jax
import jax.numpy as jnp
from jax import lax
from jax.experimental import pallas as pl
from jax.experimental.pallas import tpu as pltpu
from jax.experimental.pallas import tpu_sc as plsc

N = 10000
E = 160000
D_IN = 256
D_OUT = 256
D_H = 128
D_P = 64

NC = 2
NS = 16
DH = D_IN // NC
EPT = 10240
CHUNK = 80
NCHUNK = EPT // CHUNK
ROWS_T = 10240 // NS
NPAD = 10240
CPT = NPAD // NS


U = 4
NITER = NCHUNK // U


def _sc_body(x2, src, dst, z2, z1, agg_out, cnt_out, *scr):
    sv = scr[0:U]
    dv = scr[U:2 * U]
    rv = scr[2 * U:3 * U]
    ones_v, acc_sh, cnt_sh, sem_i, sem_g, sem_s = scr[3 * U:]
    cid = lax.axis_index("c")
    sid = lax.axis_index("s")

    def idx_copies(t, u):
        base = sid * EPT + (t * U + u) * CHUNK
        return (pltpu.make_async_copy(src.at[pl.ds(base, CHUNK)], sv[u], sem_i.at[u]),
                pltpu.make_async_copy(dst.at[pl.ds(base, CHUNK)], dv[u], sem_i.at[u]))

    for u in range(U):
        a, b = idx_copies(0, u)
        a.start()
        b.start()

    pltpu.sync_copy(z2.at[pl.ds(sid * ROWS_T, ROWS_T)],
                    acc_sh.at[pl.ds(sid * ROWS_T, ROWS_T)])

    @pl.when(cid == 0)
    def _():
        pltpu.sync_copy(z1.at[pl.ds(sid * CPT, CPT)],
                        cnt_sh.at[pl.ds(sid * CPT, CPT)])

    for j in range(CHUNK // 16):
        ones_v[pl.ds(j * 16, 16)] = jnp.ones((16,), jnp.float32)

    plsc.subcore_barrier()

    def step(t, _):
        for u in range(U):
            a, b = idx_copies(t, u)
            a.wait()
            b.wait()
        for u in range(U):
            for j in range(CHUNK // 16):
                v = sv[u][pl.ds(j * 16, 16)]
                sv[u][pl.ds(j * 16, 16)] = v * 2 + cid
        for u in range(U):
            pltpu.async_copy(x2.at[sv[u]], rv[u], sem_g.at[u])
        for u in range(U):
            pltpu.make_async_copy(x2.at[sv[u]], rv[u], sem_g.at[u]).wait()
            pltpu.async_copy(rv[u], acc_sh.at[dv[u]], sem_s.at[u], add=True)

            @pl.when(cid == 0)
            def _():
                pltpu.async_copy(ones_v, cnt_sh.at[dv[u]], sem_s.at[u], add=True)
        for u in range(U):
            pltpu.make_async_copy(rv[u], acc_sh.at[dv[u]], sem_s.at[u]).wait()

            @pl.when(cid == 0)
            def _():
                pltpu.make_async_copy(ones_v, cnt_sh.at[dv[u]], sem_s.at[u]).wait()

            @pl.when(t + 1 < NITER)
            def _():
                a, b = idx_copies(t + 1, u)
                a.start()
                b.start()

        return _

    lax.fori_loop(0, NITER, step, None)

    plsc.subcore_barrier()

    pltpu.sync_copy(acc_sh.at[pl.ds(sid * ROWS_T, ROWS_T)],
                    agg_out.at[cid, pl.ds(sid * ROWS_T, ROWS_T)])

    @pl.when(cid == 0)
    def _():
        pltpu.sync_copy(cnt_sh.at[pl.ds(sid * CPT, CPT)],
                        cnt_out.at[pl.ds(sid * CPT, CPT)])


def _sc_segment_sum(x2, src, dst):
    z2 = jnp.zeros((NPAD, DH), jnp.float32)
    z1 = jnp.zeros((NPAD,), jnp.float32)
    mesh = plsc.VectorSubcoreMesh(core_axis_name="c", subcore_axis_name="s")
    k = pl.kernel(
        _sc_body,
        out_type=(jax.ShapeDtypeStruct((NC, NPAD, DH), jnp.float32),
                  jax.ShapeDtypeStruct((NPAD,), jnp.float32)),
        mesh=mesh,
        scratch_types=(
            [pltpu.VMEM((CHUNK,), jnp.int32) for _ in range(U)]
            + [pltpu.VMEM((CHUNK,), jnp.int32) for _ in range(U)]
            + [pltpu.VMEM((CHUNK, DH), jnp.float32) for _ in range(U)]
            + [
            pltpu.VMEM((CHUNK,), jnp.float32),
            pltpu.VMEM_SHARED((NPAD, DH), jnp.float32),
            pltpu.VMEM_SHARED((NPAD,), jnp.float32),
            pltpu.SemaphoreType.DMA((U,)),
            pltpu.SemaphoreType.DMA((U,)),
            pltpu.SemaphoreType.DMA((U,)),
        ]),
    )
    return k(x2, src, dst, z2, z1)


ROW_BLK = 1000


def _dense_body(x_ref, a0_ref, a1_ref, cnt_ref, wn0_ref, wn1_ref, wr_ref,
                b_ref, gam_ref, bet_ref, wg1_ref, bg1_ref, wg2_ref, bg2_ref,
                wp_ref, bp_ref, g_ref, xp_ref):
    inv = 1.0 / jnp.maximum(cnt_ref[...], 1.0)
    m0 = a0_ref[...] * inv
    m1 = a1_ref[...] * inv
    dims = (((1,), (1,)), ((), ()))
    h = (lax.dot_general(m0, wn0_ref[...], dims, preferred_element_type=jnp.float32)
         + lax.dot_general(m1, wn1_ref[...], dims, preferred_element_type=jnp.float32)
         + lax.dot_general(x_ref[...], wr_ref[...], dims, preferred_element_type=jnp.float32)
         + b_ref[...])
    h = jnp.maximum(h, 0.0)
    mu = jnp.mean(h, axis=-1, keepdims=True)
    hc = h - mu
    var = jnp.mean(hc * hc, axis=-1, keepdims=True)
    h = hc * lax.rsqrt(var + 1e-5) * gam_ref[...] + bet_ref[...]
    hg = jnp.maximum(
        lax.dot_general(h, wg1_ref[...], dims, preferred_element_type=jnp.float32)
        + bg1_ref[...], 0.0)
    g_ref[...] = (lax.dot_general(hg, wg2_ref[...], dims,
                                  preferred_element_type=jnp.float32)
                  + bg2_ref[...])
    xp_ref[...] = jnp.maximum(
        lax.dot_general(h, wp_ref[...], dims, preferred_element_type=jnp.float32)
        + bp_ref[...], 0.0)


def _finish_body(g_ref, xp_ref, out_ref):
    g = g_ref[...]
    m = jnp.max(g, axis=0, keepdims=True)
    e = jnp.exp(g - m)
    s = jnp.sum(e, axis=0, keepdims=True)
    o = (e / s) * xp_ref[...]
    nrm = jnp.sqrt(jnp.sum(o * o, axis=1, keepdims=True))
    out_ref[...] = o / jnp.maximum(nrm, 1e-12)


def kernel(x, edge_index, W_neigh, W_root, b_sage, gamma, beta,
           Wg1, bg1, Wg2, bg2, Wp, bp):
    src = edge_index[0].astype(jnp.int32)
    dst = edge_index[1].astype(jnp.int32)
    x2 = x.reshape(2 * N, DH)

    npe = EPT - E // NS
    pad_s = (jnp.arange(npe, dtype=jnp.int32)[None, :] * 41
             + jnp.arange(NS, dtype=jnp.int32)[:, None] * 13) % N
    pad_d = (N + jnp.arange(npe, dtype=jnp.int32)[None, :]
             + jnp.zeros((NS, 1), jnp.int32))
    src_p = jnp.concatenate([src.reshape(NS, -1), pad_s], axis=1).reshape(-1)
    dst_p = jnp.concatenate([dst.reshape(NS, -1), pad_d], axis=1).reshape(-1)

    agg, cnt = _sc_segment_sum(x2, src_p, dst_p)

    grid = N // ROW_BLK
    row = lambda i: (i, 0)
    full = lambda i: (0, 0)
    g, xp = pl.pallas_call(
        _dense_body,
        grid=(grid,),
        in_specs=[
            pl.BlockSpec((ROW_BLK, D_IN), row),
            pl.BlockSpec((ROW_BLK, DH), row),
            pl.BlockSpec((ROW_BLK, DH), row),
            pl.BlockSpec((ROW_BLK, 1), row),
            pl.BlockSpec((D_OUT, DH), full),
            pl.BlockSpec((D_OUT, DH), full),
            pl.BlockSpec((D_OUT, D_IN), full),
            pl.BlockSpec((1, D_OUT), full),
            pl.BlockSpec((1, D_OUT), full),
            pl.BlockSpec((1, D_OUT), full),
            pl.BlockSpec((D_H, D_OUT), full),
            pl.BlockSpec((1, D_H), full),
            pl.BlockSpec((D_P, D_H), full),
            pl.BlockSpec((1, D_P), full),
            pl.BlockSpec((D_P, D_OUT), full),
            pl.BlockSpec((1, D_P), full),
        ],
        out_specs=[
            pl.BlockSpec((ROW_BLK, D_P), row),
            pl.BlockSpec((ROW_BLK, D_P), row),
        ],
        out_shape=[
            jax.ShapeDtypeStruct((N, D_P), jnp.float32),
            jax.ShapeDtypeStruct((N, D_P), jnp.float32),
        ],
    )(x, agg[0, :N], agg[1, :N], cnt[:N].reshape(N, 1),
      W_neigh[:, :DH], W_neigh[:, DH:], W_root,
      b_sage.reshape(1, -1), gamma.reshape(1, -1), beta.reshape(1, -1),
      Wg1, bg1.reshape(1, -1), Wg2, bg2.reshape(1, -1),
      Wp, bp.reshape(1, -1))

    out = pl.pallas_call(
        _finish_body,
        out_shape=jax.ShapeDtypeStruct((N, D_P), jnp.float32),
    )(g, xp)
    return out


# speedup vs baseline: 7.6213x; 1.2293x over previous
import functools

import jax
import jax.numpy as jnp
from jax import lax
from jax.experimental import pallas as pl
from jax.experimental.pallas import tpu as pltpu
from jax.experimental.pallas import tpu_sc as plsc

N = 10000
E = 160000
D_IN = 256
D_OUT = 256
D_H = 128
D_P = 64

NC = 2
NS = 16
DH = D_IN // NC
EPT = 10240
CHUNK = 80
NCHUNK = EPT // CHUNK
ROWS_T = 10240 // NS
NPAD = 10240
CPT = NPAD // NS


U = 4
NITER = NCHUNK // U


def _sc_body(x2, src, dst, z2, z1, agg_out, cnt_out, *scr):
    sv = scr[0:U]
    dv = scr[U:2 * U]
    rv = scr[2 * U:3 * U]
    ones_v, acc_sh, cnt_sh, sem_src, sem_dst, sem_g, sem_s = scr[3 * U:]
    cid = lax.axis_index("c")
    sid = lax.axis_index("s")

    def src_copy(t, u):
        base = sid * EPT + (t * U + u) * CHUNK
        return pltpu.make_async_copy(src.at[pl.ds(base, CHUNK)], sv[u],
                                     sem_src.at[u])

    def dst_copy(t, u):
        base = sid * EPT + (t * U + u) * CHUNK
        return pltpu.make_async_copy(dst.at[pl.ds(base, CHUNK)], dv[u],
                                     sem_dst.at[u])

    for u in range(U):
        src_copy(0, u).start()

    pltpu.sync_copy(z2.at[pl.ds(sid * ROWS_T, ROWS_T)],
                    acc_sh.at[pl.ds(sid * ROWS_T, ROWS_T)])

    @pl.when(cid == 0)
    def _():
        pltpu.sync_copy(z1.at[pl.ds(sid * CPT, CPT)],
                        cnt_sh.at[pl.ds(sid * CPT, CPT)])

    for j in range(CHUNK // 16):
        ones_v[pl.ds(j * 16, 16)] = jnp.ones((16,), jnp.float32)

    plsc.subcore_barrier()

    def step(t, _):
        for u in range(U):
            @pl.when(t > 0)
            def _():
                pltpu.make_async_copy(rv[u], acc_sh.at[dv[u]],
                                      sem_s.at[u]).wait()

                @pl.when(cid == 0)
                def _():
                    pltpu.make_async_copy(ones_v, cnt_sh.at[dv[u]],
                                          sem_s.at[u]).wait()

            dst_copy(t, u).start()
            src_copy(t, u).wait()
            for j in range(CHUNK // 16):
                v = sv[u][pl.ds(j * 16, 16)]
                sv[u][pl.ds(j * 16, 16)] = v * 2 + cid
            pltpu.async_copy(x2.at[sv[u]], rv[u], sem_g.at[u])

        for u in range(U):
            pltpu.make_async_copy(x2.at[sv[u]], rv[u], sem_g.at[u]).wait()

            @pl.when(t + 1 < NITER)
            def _():
                src_copy(t + 1, u).start()

            dst_copy(t, u).wait()
            pltpu.async_copy(rv[u], acc_sh.at[dv[u]], sem_s.at[u], add=True)

            @pl.when(cid == 0)
            def _():
                pltpu.async_copy(ones_v, cnt_sh.at[dv[u]], sem_s.at[u],
                                 add=True)

        return _

    lax.fori_loop(0, NITER, step, None)

    for u in range(U):
        pltpu.make_async_copy(rv[u], acc_sh.at[dv[u]], sem_s.at[u]).wait()

        @pl.when(cid == 0)
        def _():
            pltpu.make_async_copy(ones_v, cnt_sh.at[dv[u]], sem_s.at[u]).wait()

    plsc.subcore_barrier()

    pltpu.sync_copy(acc_sh.at[pl.ds(sid * ROWS_T, ROWS_T)],
                    agg_out.at[cid, pl.ds(sid * ROWS_T, ROWS_T)])

    @pl.when(cid == 0)
    def _():
        pltpu.sync_copy(cnt_sh.at[pl.ds(sid * CPT, CPT)],
                        cnt_out.at[pl.ds(sid * CPT, CPT)])


def _sc_segment_sum(x2, src, dst):
    z2 = jnp.zeros((NPAD, DH), jnp.float32)
    z1 = jnp.zeros((NPAD,), jnp.float32)
    mesh = plsc.VectorSubcoreMesh(core_axis_name="c", subcore_axis_name="s")
    k = pl.kernel(
        _sc_body,
        out_type=(jax.ShapeDtypeStruct((NC, NPAD, DH), jnp.float32),
                  jax.ShapeDtypeStruct((NPAD,), jnp.float32)),
        mesh=mesh,
        scratch_types=(
            [pltpu.VMEM((CHUNK,), jnp.int32) for _ in range(U)]
            + [pltpu.VMEM((CHUNK,), jnp.int32) for _ in range(U)]
            + [pltpu.VMEM((CHUNK, DH), jnp.float32) for _ in range(U)]
            + [
            pltpu.VMEM((CHUNK,), jnp.float32),
            pltpu.VMEM_SHARED((NPAD, DH), jnp.float32),
            pltpu.VMEM_SHARED((NPAD,), jnp.float32),
            pltpu.SemaphoreType.DMA((U,)),
            pltpu.SemaphoreType.DMA((U,)),
            pltpu.SemaphoreType.DMA((U,)),
            pltpu.SemaphoreType.DMA((U,)),
        ]),
    )
    return k(x2, src, dst, z2, z1)


ROW_BLK = 1000


def _dense_body(x_ref, a0_ref, a1_ref, cnt_ref, wn0_ref, wn1_ref, wr_ref,
                b_ref, gam_ref, bet_ref, wg1_ref, bg1_ref, wg2_ref, bg2_ref,
                wp_ref, bp_ref, g_ref, xp_ref):
    inv = 1.0 / jnp.maximum(cnt_ref[...], 1.0)
    m0 = a0_ref[...] * inv
    m1 = a1_ref[...] * inv
    dims = (((1,), (1,)), ((), ()))
    h = (lax.dot_general(m0, wn0_ref[...], dims, preferred_element_type=jnp.float32)
         + lax.dot_general(m1, wn1_ref[...], dims, preferred_element_type=jnp.float32)
         + lax.dot_general(x_ref[...], wr_ref[...], dims, preferred_element_type=jnp.float32)
         + b_ref[...])
    h = jnp.maximum(h, 0.0)
    mu = jnp.mean(h, axis=-1, keepdims=True)
    hc = h - mu
    var = jnp.mean(hc * hc, axis=-1, keepdims=True)
    h = hc * lax.rsqrt(var + 1e-5) * gam_ref[...] + bet_ref[...]
    hg = jnp.maximum(
        lax.dot_general(h, wg1_ref[...], dims, preferred_element_type=jnp.float32)
        + bg1_ref[...], 0.0)
    g_ref[...] = (lax.dot_general(hg, wg2_ref[...], dims,
                                  preferred_element_type=jnp.float32)
                  + bg2_ref[...])
    xp_ref[...] = jnp.maximum(
        lax.dot_general(h, wp_ref[...], dims, preferred_element_type=jnp.float32)
        + bp_ref[...], 0.0)


def _finish_body(g_ref, xp_ref, out_ref):
    g = g_ref[...]
    m = jnp.max(g, axis=0, keepdims=True)
    e = jnp.exp(g - m)
    s = jnp.sum(e, axis=0, keepdims=True)
    o = (e / s) * xp_ref[...]
    nrm = jnp.sqrt(jnp.sum(o * o, axis=1, keepdims=True))
    out_ref[...] = o / jnp.maximum(nrm, 1e-12)


def kernel(x, edge_index, W_neigh, W_root, b_sage, gamma, beta,
           Wg1, bg1, Wg2, bg2, Wp, bp):
    src = edge_index[0].astype(jnp.int32)
    dst = edge_index[1].astype(jnp.int32)
    x2 = x.reshape(2 * N, DH)

    npe = EPT - E // NS
    pad_s = (jnp.arange(npe, dtype=jnp.int32)[None, :] * 41
             + jnp.arange(NS, dtype=jnp.int32)[:, None] * 13) % N
    pad_d = (N + jnp.arange(npe, dtype=jnp.int32)[None, :]
             + jnp.zeros((NS, 1), jnp.int32))
    src_p = jnp.concatenate([src.reshape(NS, -1), pad_s], axis=1).reshape(-1)
    dst_p = jnp.concatenate([dst.reshape(NS, -1), pad_d], axis=1).reshape(-1)

    agg, cnt = _sc_segment_sum(x2, src_p, dst_p)

    grid = N // ROW_BLK
    row = lambda i: (i, 0)
    full = lambda i: (0, 0)
    g, xp = pl.pallas_call(
        _dense_body,
        grid=(grid,),
        in_specs=[
            pl.BlockSpec((ROW_BLK, D_IN), row),
            pl.BlockSpec((ROW_BLK, DH), row),
            pl.BlockSpec((ROW_BLK, DH), row),
            pl.BlockSpec((ROW_BLK, 1), row),
            pl.BlockSpec((D_OUT, DH), full),
            pl.BlockSpec((D_OUT, DH), full),
            pl.BlockSpec((D_OUT, D_IN), full),
            pl.BlockSpec((1, D_OUT), full),
            pl.BlockSpec((1, D_OUT), full),
            pl.BlockSpec((1, D_OUT), full),
            pl.BlockSpec((D_H, D_OUT), full),
            pl.BlockSpec((1, D_H), full),
            pl.BlockSpec((D_P, D_H), full),
            pl.BlockSpec((1, D_P), full),
            pl.BlockSpec((D_P, D_OUT), full),
            pl.BlockSpec((1, D_P), full),
        ],
        out_specs=[
            pl.BlockSpec((ROW_BLK, D_P), row),
            pl.BlockSpec((ROW_BLK, D_P), row),
        ],
        out_shape=[
            jax.ShapeDtypeStruct((N, D_P), jnp.float32),
            jax.ShapeDtypeStruct((N, D_P), jnp.float32),
        ],
    )(x, agg[0, :N], agg[1, :N], cnt[:N].reshape(N, 1),
      W_neigh[:, :DH], W_neigh[:, DH:], W_root,
      b_sage.reshape(1, -1), gamma.reshape(1, -1), beta.reshape(1, -1),
      Wg1, bg1.reshape(1, -1), Wg2, bg2.reshape(1, -1),
      Wp, bp.reshape(1, -1))

    out = pl.pallas_call(
        _finish_body,
        out_shape=jax.ShapeDtypeStruct((N, D_P), jnp.float32),
    )(g, xp)
    return out


# speedup vs baseline: 8.0158x; 1.0518x over previous
import functools

import jax
import jax.numpy as jnp
from jax import lax
from jax.experimental import pallas as pl
from jax.experimental.pallas import tpu as pltpu
from jax.experimental.pallas import tpu_sc as plsc

N = 10000
E = 160000
D_IN = 256
D_OUT = 256
D_H = 128
D_P = 64

NC = 2
NS = 16
DH = D_IN // NC
EPT = 10240
CHUNK = 80
NCHUNK = EPT // CHUNK
ROWS_T = 10240 // NS
NPAD = 10240
CPT = NPAD // NS


U = 4
NITER = NCHUNK // U


def _sc_body(x2, src, dst, z2, z1, agg_out, cnt_out, *scr):
    sv = scr[0:U]
    dv = scr[U:2 * U]
    rv = scr[2 * U:3 * U]
    ones_v, acc_sh, cnt_sh, sem_src, sem_dst, sem_g, sem_s = scr[3 * U:]
    cid = lax.axis_index("c")
    sid = lax.axis_index("s")

    def src_copy(t, u):
        base = sid * EPT + (t * U + u) * CHUNK
        return pltpu.make_async_copy(src.at[pl.ds(base, CHUNK)], sv[u],
                                     sem_src.at[u])

    def dst_copy(t, u):
        base = sid * EPT + (t * U + u) * CHUNK
        return pltpu.make_async_copy(dst.at[pl.ds(base, CHUNK)], dv[u],
                                     sem_dst.at[u])

    for u in range(U):
        src_copy(0, u).start()

    pltpu.sync_copy(z2.at[pl.ds(sid * ROWS_T, ROWS_T)],
                    acc_sh.at[pl.ds(sid * ROWS_T, ROWS_T)])

    @pl.when(cid == 0)
    def _():
        pltpu.sync_copy(z1.at[pl.ds(sid * CPT, CPT)],
                        cnt_sh.at[pl.ds(sid * CPT, CPT)])

    for j in range(CHUNK // 16):
        ones_v[pl.ds(j * 16, 16)] = jnp.ones((16,), jnp.float32)

    plsc.subcore_barrier()

    def step(t, _):
        for u in range(U):
            @pl.when(t > 0)
            def _():
                pltpu.make_async_copy(rv[u], acc_sh.at[dv[u]],
                                      sem_s.at[u]).wait()

                @pl.when(cid == 0)
                def _():
                    pltpu.make_async_copy(ones_v, cnt_sh.at[dv[u]],
                                          sem_s.at[u]).wait()

            dst_copy(t, u).start()
            src_copy(t, u).wait()
            for j in range(CHUNK // 16):
                v = sv[u][pl.ds(j * 16, 16)]
                sv[u][pl.ds(j * 16, 16)] = v * 2 + cid
            pltpu.async_copy(x2.at[sv[u]], rv[u], sem_g.at[u])

        for u in range(U):
            pltpu.make_async_copy(x2.at[sv[u]], rv[u], sem_g.at[u]).wait()

            @pl.when(t + 1 < NITER)
            def _():
                src_copy(t + 1, u).start()

            dst_copy(t, u).wait()
            pltpu.async_copy(rv[u], acc_sh.at[dv[u]], sem_s.at[u], add=True)

            @pl.when(cid == 0)
            def _():
                pltpu.async_copy(ones_v, cnt_sh.at[dv[u]], sem_s.at[u],
                                 add=True)

        return _

    lax.fori_loop(0, NITER, step, None)

    for u in range(U):
        pltpu.make_async_copy(rv[u], acc_sh.at[dv[u]], sem_s.at[u]).wait()

        @pl.when(cid == 0)
        def _():
            pltpu.make_async_copy(ones_v, cnt_sh.at[dv[u]], sem_s.at[u]).wait()

    plsc.subcore_barrier()

    pltpu.sync_copy(acc_sh.at[pl.ds(sid * ROWS_T, ROWS_T)],
                    agg_out.at[cid, pl.ds(sid * ROWS_T, ROWS_T)])

    @pl.when(cid == 0)
    def _():
        pltpu.sync_copy(cnt_sh.at[pl.ds(sid * CPT, CPT)],
                        cnt_out.at[pl.ds(sid * CPT, CPT)])


def _sc_segment_sum(x2, src, dst):
    z2 = jnp.zeros((NPAD, DH), jnp.float32)
    z1 = jnp.zeros((NPAD,), jnp.float32)
    mesh = plsc.VectorSubcoreMesh(core_axis_name="c", subcore_axis_name="s")
    k = pl.kernel(
        _sc_body,
        out_type=(jax.ShapeDtypeStruct((NC, NPAD, DH), jnp.float32),
                  jax.ShapeDtypeStruct((NPAD,), jnp.float32)),
        mesh=mesh,
        scratch_types=(
            [pltpu.VMEM((CHUNK,), jnp.int32) for _ in range(U)]
            + [pltpu.VMEM((CHUNK,), jnp.int32) for _ in range(U)]
            + [pltpu.VMEM((CHUNK, DH), jnp.float32) for _ in range(U)]
            + [
            pltpu.VMEM((CHUNK,), jnp.float32),
            pltpu.VMEM_SHARED((NPAD, DH), jnp.float32),
            pltpu.VMEM_SHARED((NPAD,), jnp.float32),
            pltpu.SemaphoreType.DMA((U,)),
            pltpu.SemaphoreType.DMA((U,)),
            pltpu.SemaphoreType.DMA((U,)),
            pltpu.SemaphoreType.DMA((U,)),
        ]),
    )
    return k(x2, src, dst, z2, z1)


ROW_BLK = 1000


def _root_body(x_ref, wr_ref, r_ref):
    dims = (((1,), (1,)), ((), ()))
    r_ref[...] = lax.dot_general(x_ref[...], wr_ref[...], dims,
                                 preferred_element_type=jnp.float32)


def _dense_body(r_ref, a_ref, cnt_ref, wn0_ref, wn1_ref,
                b_ref, gam_ref, bet_ref, wg1_ref, bg1_ref, wg2_ref, bg2_ref,
                wp_ref, bp_ref, g_ref, xp_ref):
    inv = 1.0 / jnp.maximum(cnt_ref[...], 1.0)
    m0 = a_ref[0] * inv
    m1 = a_ref[1] * inv
    dims = (((1,), (1,)), ((), ()))
    h = (lax.dot_general(m0, wn0_ref[...], dims, preferred_element_type=jnp.float32)
         + lax.dot_general(m1, wn1_ref[...], dims, preferred_element_type=jnp.float32)
         + r_ref[...]
         + b_ref[...])
    h = jnp.maximum(h, 0.0)
    mu = jnp.mean(h, axis=-1, keepdims=True)
    hc = h - mu
    var = jnp.mean(hc * hc, axis=-1, keepdims=True)
    h = hc * lax.rsqrt(var + 1e-5) * gam_ref[...] + bet_ref[...]
    hg = jnp.maximum(
        lax.dot_general(h, wg1_ref[...], dims, preferred_element_type=jnp.float32)
        + bg1_ref[...], 0.0)
    g_ref[...] = (lax.dot_general(hg, wg2_ref[...], dims,
                                  preferred_element_type=jnp.float32)
                  + bg2_ref[...])
    xp_ref[...] = jnp.maximum(
        lax.dot_general(h, wp_ref[...], dims, preferred_element_type=jnp.float32)
        + bp_ref[...], 0.0)


def _finish_body(g_ref, xp_ref, out_ref):
    g = g_ref[...]
    m = jnp.max(g, axis=0, keepdims=True)
    e = jnp.exp(g - m)
    s = jnp.sum(e, axis=0, keepdims=True)
    o = (e / s) * xp_ref[...]
    nrm = jnp.sqrt(jnp.sum(o * o, axis=1, keepdims=True))
    out_ref[...] = o / jnp.maximum(nrm, 1e-12)


def kernel(x, edge_index, W_neigh, W_root, b_sage, gamma, beta,
           Wg1, bg1, Wg2, bg2, Wp, bp):
    src = edge_index[0].astype(jnp.int32)
    dst = edge_index[1].astype(jnp.int32)
    x2 = x.reshape(2 * N, DH)

    npe = EPT - E // NS
    pad_s = (jnp.arange(npe, dtype=jnp.int32)[None, :] * 41
             + jnp.arange(NS, dtype=jnp.int32)[:, None] * 13) % N
    pad_d = (N + jnp.arange(npe, dtype=jnp.int32)[None, :]
             + jnp.zeros((NS, 1), jnp.int32))
    src_p = jnp.concatenate([src.reshape(NS, -1), pad_s], axis=1).reshape(-1)
    dst_p = jnp.concatenate([dst.reshape(NS, -1), pad_d], axis=1).reshape(-1)

    grid = N // ROW_BLK
    row = lambda i: (i, 0)
    full = lambda i: (0, 0)

    r = pl.pallas_call(
        _root_body,
        grid=(grid,),
        in_specs=[
            pl.BlockSpec((ROW_BLK, D_IN), row),
            pl.BlockSpec((D_OUT, D_IN), full),
        ],
        out_specs=pl.BlockSpec((ROW_BLK, D_OUT), row),
        out_shape=jax.ShapeDtypeStruct((N, D_OUT), jnp.float32),
    )(x, W_root)

    agg, cnt = _sc_segment_sum(x2, src_p, dst_p)

    g, xp = pl.pallas_call(
        _dense_body,
        grid=(grid,),
        in_specs=[
            pl.BlockSpec((ROW_BLK, D_OUT), row),
            pl.BlockSpec((NC, ROW_BLK, DH), lambda i: (0, i, 0)),
            pl.BlockSpec((ROW_BLK, 1), row),
            pl.BlockSpec((D_OUT, DH), full),
            pl.BlockSpec((D_OUT, DH), full),
            pl.BlockSpec((1, D_OUT), full),
            pl.BlockSpec((1, D_OUT), full),
            pl.BlockSpec((1, D_OUT), full),
            pl.BlockSpec((D_H, D_OUT), full),
            pl.BlockSpec((1, D_H), full),
            pl.BlockSpec((D_P, D_H), full),
            pl.BlockSpec((1, D_P), full),
            pl.BlockSpec((D_P, D_OUT), full),
            pl.BlockSpec((1, D_P), full),
        ],
        out_specs=[
            pl.BlockSpec((ROW_BLK, D_P), row),
            pl.BlockSpec((ROW_BLK, D_P), row),
        ],
        out_shape=[
            jax.ShapeDtypeStruct((N, D_P), jnp.float32),
            jax.ShapeDtypeStruct((N, D_P), jnp.float32),
        ],
    )(r, agg, cnt.reshape(NPAD, 1),
      W_neigh[:, :DH], W_neigh[:, DH:],
      b_sage.reshape(1, -1), gamma.reshape(1, -1), beta.reshape(1, -1),
      Wg1, bg1.reshape(1, -1), Wg2, bg2.reshape(1, -1),
      Wp, bp.reshape(1, -1))

    out = pl.pallas_call(
        _finish_body,
        out_shape=jax.ShapeDtypeStruct((N, D_P), jnp.float32),
    )(g, xp)
    return out


# speedup vs baseline: 8.0592x; 1.0054x over previous
import functools

import jax
import jax.numpy as jnp
from jax import lax
from jax.experimental import pallas as pl
from jax.experimental.pallas import tpu as pltpu
from jax.experimental.pallas import tpu_sc as plsc

N = 10000
E = 160000
D_IN = 256
D_OUT = 256
D_H = 128
D_P = 64

NC = 2
NS = 16
DH = D_IN // NC
EPT = 10240
CHUNK = 64
NCHUNK = EPT // CHUNK
ROWS_T = 10240 // NS
NPAD = 10240
CPT = NPAD // NS


U = 5
NITER = NCHUNK // U


def _sc_body(x2, src, dst, z2, z1, agg_out, cnt_out, *scr):
    sv = scr[0:U]
    dv = scr[U:2 * U]
    rv = scr[2 * U:3 * U]
    ones_v, acc_sh, cnt_sh, sem_src, sem_dst, sem_g, sem_s = scr[3 * U:]
    cid = lax.axis_index("c")
    sid = lax.axis_index("s")

    def src_copy(t, u):
        base = sid * EPT + (t * U + u) * CHUNK
        return pltpu.make_async_copy(src.at[pl.ds(base, CHUNK)], sv[u],
                                     sem_src.at[u])

    def dst_copy(t, u):
        base = sid * EPT + (t * U + u) * CHUNK
        return pltpu.make_async_copy(dst.at[pl.ds(base, CHUNK)], dv[u],
                                     sem_dst.at[u])

    for u in range(U):
        src_copy(0, u).start()

    pltpu.sync_copy(z2.at[pl.ds(sid * ROWS_T, ROWS_T)],
                    acc_sh.at[pl.ds(sid * ROWS_T, ROWS_T)])

    @pl.when(cid == 0)
    def _():
        pltpu.sync_copy(z1.at[pl.ds(sid * CPT, CPT)],
                        cnt_sh.at[pl.ds(sid * CPT, CPT)])

    for j in range(CHUNK // 16):
        ones_v[pl.ds(j * 16, 16)] = jnp.ones((16,), jnp.float32)

    plsc.subcore_barrier()

    def step(t, _):
        for u in range(U):
            @pl.when(t > 0)
            def _():
                pltpu.make_async_copy(rv[u], acc_sh.at[dv[u]],
                                      sem_s.at[u]).wait()

                @pl.when(cid == 0)
                def _():
                    pltpu.make_async_copy(ones_v, cnt_sh.at[dv[u]],
                                          sem_s.at[u]).wait()

            dst_copy(t, u).start()
            src_copy(t, u).wait()
            for j in range(CHUNK // 16):
                v = sv[u][pl.ds(j * 16, 16)]
                sv[u][pl.ds(j * 16, 16)] = v * 2 + cid
            pltpu.async_copy(x2.at[sv[u]], rv[u], sem_g.at[u])

        for u in range(U):
            pltpu.make_async_copy(x2.at[sv[u]], rv[u], sem_g.at[u]).wait()

            @pl.when(t + 1 < NITER)
            def _():
                src_copy(t + 1, u).start()

            dst_copy(t, u).wait()
            pltpu.async_copy(rv[u], acc_sh.at[dv[u]], sem_s.at[u], add=True)

            @pl.when(cid == 0)
            def _():
                pltpu.async_copy(ones_v, cnt_sh.at[dv[u]], sem_s.at[u],
                                 add=True)

        return _

    lax.fori_loop(0, NITER, step, None)

    for u in range(U):
        pltpu.make_async_copy(rv[u], acc_sh.at[dv[u]], sem_s.at[u]).wait()

        @pl.when(cid == 0)
        def _():
            pltpu.make_async_copy(ones_v, cnt_sh.at[dv[u]], sem_s.at[u]).wait()

    plsc.subcore_barrier()

    pltpu.sync_copy(acc_sh.at[pl.ds(sid * ROWS_T, ROWS_T)],
                    agg_out.at[cid, pl.ds(sid * ROWS_T, ROWS_T)])

    @pl.when(cid == 0)
    def _():
        pltpu.sync_copy(cnt_sh.at[pl.ds(sid * CPT, CPT)],
                        cnt_out.at[pl.ds(sid * CPT, CPT)])


def _sc_segment_sum(x2, src, dst):
    z2 = jnp.zeros((NPAD, DH), jnp.float32)
    z1 = jnp.zeros((NPAD,), jnp.float32)
    mesh = plsc.VectorSubcoreMesh(core_axis_name="c", subcore_axis_name="s")
    k = pl.kernel(
        _sc_body,
        out_type=(jax.ShapeDtypeStruct((NC, NPAD, DH), jnp.float32),
                  jax.ShapeDtypeStruct((NPAD,), jnp.float32)),
        mesh=mesh,
        scratch_types=(
            [pltpu.VMEM((CHUNK,), jnp.int32) for _ in range(U)]
            + [pltpu.VMEM((CHUNK,), jnp.int32) for _ in range(U)]
            + [pltpu.VMEM((CHUNK, DH), jnp.float32) for _ in range(U)]
            + [
            pltpu.VMEM((CHUNK,), jnp.float32),
            pltpu.VMEM_SHARED((NPAD, DH), jnp.float32),
            pltpu.VMEM_SHARED((NPAD,), jnp.float32),
            pltpu.SemaphoreType.DMA((U,)),
            pltpu.SemaphoreType.DMA((U,)),
            pltpu.SemaphoreType.DMA((U,)),
            pltpu.SemaphoreType.DMA((U,)),
        ]),
    )
    return k(x2, src, dst, z2, z1)


ROW_BLK = 1000


def _root_body(x_ref, wr_ref, r_ref):
    dims = (((1,), (1,)), ((), ()))
    r_ref[...] = lax.dot_general(x_ref[...], wr_ref[...], dims,
                                 preferred_element_type=jnp.float32)


def _dense_body(r_ref, a_ref, cnt_ref, wn0_ref, wn1_ref,
                b_ref, gam_ref, bet_ref, wg1_ref, bg1_ref, wg2_ref, bg2_ref,
                wp_ref, bp_ref, g_ref, xp_ref):
    inv = 1.0 / jnp.maximum(cnt_ref[...], 1.0)
    m0 = a_ref[0] * inv
    m1 = a_ref[1] * inv
    dims = (((1,), (1,)), ((), ()))
    h = (lax.dot_general(m0, wn0_ref[...], dims, preferred_element_type=jnp.float32)
         + lax.dot_general(m1, wn1_ref[...], dims, preferred_element_type=jnp.float32)
         + r_ref[...]
         + b_ref[...])
    h = jnp.maximum(h, 0.0)
    mu = jnp.mean(h, axis=-1, keepdims=True)
    hc = h - mu
    var = jnp.mean(hc * hc, axis=-1, keepdims=True)
    h = hc * lax.rsqrt(var + 1e-5) * gam_ref[...] + bet_ref[...]
    hg = jnp.maximum(
        lax.dot_general(h, wg1_ref[...], dims, preferred_element_type=jnp.float32)
        + bg1_ref[...], 0.0)
    g_ref[...] = (lax.dot_general(hg, wg2_ref[...], dims,
                                  preferred_element_type=jnp.float32)
                  + bg2_ref[...])
    xp_ref[...] = jnp.maximum(
        lax.dot_general(h, wp_ref[...], dims, preferred_element_type=jnp.float32)
        + bp_ref[...], 0.0)


def _finish_body(g_ref, xp_ref, out_ref):
    g = g_ref[...]
    m = jnp.max(g, axis=0, keepdims=True)
    e = jnp.exp(g - m)
    s = jnp.sum(e, axis=0, keepdims=True)
    o = (e / s) * xp_ref[...]
    nrm = jnp.sqrt(jnp.sum(o * o, axis=1, keepdims=True))
    out_ref[...] = o / jnp.maximum(nrm, 1e-12)


def kernel(x, edge_index, W_neigh, W_root, b_sage, gamma, beta,
           Wg1, bg1, Wg2, bg2, Wp, bp):
    src = edge_index[0].astype(jnp.int32)
    dst = edge_index[1].astype(jnp.int32)
    x2 = x.reshape(2 * N, DH)

    npe = EPT - E // NS
    pad_s = (jnp.arange(npe, dtype=jnp.int32)[None, :] * 41
             + jnp.arange(NS, dtype=jnp.int32)[:, None] * 13) % N
    pad_d = (N + jnp.arange(npe, dtype=jnp.int32)[None, :]
             + jnp.zeros((NS, 1), jnp.int32))
    src_p = jnp.concatenate([src.reshape(NS, -1), pad_s], axis=1).reshape(-1)
    dst_p = jnp.concatenate([dst.reshape(NS, -1), pad_d], axis=1).reshape(-1)

    grid = N // ROW_BLK
    row = lambda i: (i, 0)
    full = lambda i: (0, 0)

    r = pl.pallas_call(
        _root_body,
        grid=(grid,),
        in_specs=[
            pl.BlockSpec((ROW_BLK, D_IN), row),
            pl.BlockSpec((D_OUT, D_IN), full),
        ],
        out_specs=pl.BlockSpec((ROW_BLK, D_OUT), row),
        out_shape=jax.ShapeDtypeStruct((N, D_OUT), jnp.float32),
    )(x, W_root)

    agg, cnt = _sc_segment_sum(x2, src_p, dst_p)

    g, xp = pl.pallas_call(
        _dense_body,
        grid=(grid,),
        in_specs=[
            pl.BlockSpec((ROW_BLK, D_OUT), row),
            pl.BlockSpec((NC, ROW_BLK, DH), lambda i: (0, i, 0)),
            pl.BlockSpec((ROW_BLK, 1), row),
            pl.BlockSpec((D_OUT, DH), full),
            pl.BlockSpec((D_OUT, DH), full),
            pl.BlockSpec((1, D_OUT), full),
            pl.BlockSpec((1, D_OUT), full),
            pl.BlockSpec((1, D_OUT), full),
            pl.BlockSpec((D_H, D_OUT), full),
            pl.BlockSpec((1, D_H), full),
            pl.BlockSpec((D_P, D_H), full),
            pl.BlockSpec((1, D_P), full),
            pl.BlockSpec((D_P, D_OUT), full),
            pl.BlockSpec((1, D_P), full),
        ],
        out_specs=[
            pl.BlockSpec((ROW_BLK, D_P), row),
            pl.BlockSpec((ROW_BLK, D_P), row),
        ],
        out_shape=[
            jax.ShapeDtypeStruct((N, D_P), jnp.float32),
            jax.ShapeDtypeStruct((N, D_P), jnp.float32),
        ],
    )(r, agg, cnt.reshape(NPAD, 1),
      W_neigh[:, :DH], W_neigh[:, DH:],
      b_sage.reshape(1, -1), gamma.reshape(1, -1), beta.reshape(1, -1),
      Wg1, bg1.reshape(1, -1), Wg2, bg2.reshape(1, -1),
      Wp, bp.reshape(1, -1))

    out = pl.pallas_call(
        _finish_body,
        out_shape=jax.ShapeDtypeStruct((N, D_P), jnp.float32),
    )(g, xp)
    return out
